# Initial kernel scaffold; baseline (speedup 1.0000x reference)
import jax
import jax.numpy as jnp
from jax.experimental import pallas as pl


def kernel(x, edge_index, W1, a1s, a1d, b1, W2, a2s, a2d, b2, W3, a3s, a3d, b3, Wc, bc):
    raise NotImplementedError("write your pallas kernel here")


# speedup vs baseline: 45.9498x; 45.9498x over previous
import functools

import jax
import jax.numpy as jnp
from jax import lax
from jax.experimental import pallas as pl
from jax.experimental.pallas import tpu as pltpu
from jax.experimental.pallas import tpu_sc as plsc

N = 100000
NP = 102400
RB = 2048
GRID = NP // RB
NEG = 0.2
EPS = 1e-16

NC, NS = 2, 16
NW = NC * NS
EB = 128
SUBROWS = NP // NS


def _prep1_body(x_ref, w_ref, a_ref, table_ref, cvec_ref, mx_ref):
    i = pl.program_id(0)
    h = jnp.dot(x_ref[...], w_ref[...], preferred_element_type=jnp.float32)
    sd = jnp.dot(h, a_ref[...], preferred_element_type=jnp.float32)
    table_ref[...] = jnp.concatenate(
        [sd[:, 0:1], h, sd[:, 1:2], jnp.zeros((RB, 2), jnp.float32)], axis=1)
    bs = jnp.max(sd[:, 0])
    bd = jnp.max(sd[:, 1])
    ps = jnp.where(i == 0, -jnp.inf, mx_ref[0])
    pd = jnp.where(i == 0, -jnp.inf, mx_ref[1])
    mx_ref[0] = jnp.maximum(ps, bs)
    mx_ref[1] = jnp.maximum(pd, bd)

    @pl.when(i == GRID - 1)
    def _():
        t = mx_ref[0] + mx_ref[1]
        c = jnp.maximum(t, NEG * t)
        cvec_ref[...] = jnp.full((8, 128), c, jnp.float32)


def _prep1_call(xp, w1, a1):
    return pl.pallas_call(
        _prep1_body,
        grid=(GRID,),
        in_specs=[
            pl.BlockSpec((RB, 34), lambda i: (i, 0)),
            pl.BlockSpec((34, 4), lambda i: (0, 0)),
            pl.BlockSpec((4, 2), lambda i: (0, 0)),
        ],
        out_specs=[
            pl.BlockSpec((RB, 8), lambda i: (i, 0)),
            pl.BlockSpec((8, 128), lambda i: (0, 0)),
        ],
        out_shape=[
            jax.ShapeDtypeStruct((NP, 8), jnp.float32),
            jax.ShapeDtypeStruct((8, 128), jnp.float32),
        ],
        scratch_shapes=[pltpu.SMEM((2,), jnp.float32)],
    )(xp, w1, a1)


def _mid_body(acc_ref, b_ref, w_ref, a_ref, h_ref, table_ref, cvec_ref,
              mx_ref, *, fn, pout):
    i = pl.program_id(0)
    a = acc_ref[0] + acc_ref[1]
    o = a[:, 1:5] / (a[:, 0:1] + EPS) + b_ref[...]
    hl = jnp.tanh(o)
    h_ref[...] = hl
    hn = jnp.dot(hl, w_ref[...], preferred_element_type=jnp.float32)
    sd = jnp.dot(hn, a_ref[...], preferred_element_type=jnp.float32)
    parts = [sd[:, 0:1], hn, sd[:, 1:2]]
    if pout > fn + 2:
        parts.append(jnp.zeros((RB, pout - fn - 2), jnp.float32))
    table_ref[...] = jnp.concatenate(parts, axis=1)
    bs = jnp.max(sd[:, 0])
    bd = jnp.max(sd[:, 1])
    ps = jnp.where(i == 0, -jnp.inf, mx_ref[0])
    pd = jnp.where(i == 0, -jnp.inf, mx_ref[1])
    mx_ref[0] = jnp.maximum(ps, bs)
    mx_ref[1] = jnp.maximum(pd, bd)

    @pl.when(i == GRID - 1)
    def _():
        t = mx_ref[0] + mx_ref[1]
        c = jnp.maximum(t, NEG * t)
        cvec_ref[...] = jnp.full((8, 128), c, jnp.float32)


def _mid_call(acc, b, w, a, fn, pout):
    return pl.pallas_call(
        functools.partial(_mid_body, fn=fn, pout=pout),
        grid=(GRID,),
        in_specs=[
            pl.BlockSpec((NC, RB, 8), lambda i: (0, i, 0)),
            pl.BlockSpec((1, 4), lambda i: (0, 0)),
            pl.BlockSpec((4, fn), lambda i: (0, 0)),
            pl.BlockSpec((fn, 2), lambda i: (0, 0)),
        ],
        out_specs=[
            pl.BlockSpec((RB, 4), lambda i: (i, 0)),
            pl.BlockSpec((RB, pout), lambda i: (i, 0)),
            pl.BlockSpec((8, 128), lambda i: (0, 0)),
        ],
        out_shape=[
            jax.ShapeDtypeStruct((NP, 4), jnp.float32),
            jax.ShapeDtypeStruct((NP, pout), jnp.float32),
            jax.ShapeDtypeStruct((8, 128), jnp.float32),
        ],
        scratch_shapes=[pltpu.SMEM((2,), jnp.float32)],
    )(acc, b, w, a)


def _final_body(acc_ref, b_ref, wc_ref, bc_ref, h_ref, out_ref):
    a = acc_ref[0] + acc_ref[1]
    o = a[:, 1:3] / (a[:, 0:1] + EPS) + b_ref[...]
    h3 = jnp.tanh(o)
    h_ref[...] = h3
    out_ref[...] = (
        jnp.dot(h3, wc_ref[...], preferred_element_type=jnp.float32)
        + bc_ref[...])


def _final_call(acc, b3, wc, bc):
    return pl.pallas_call(
        _final_body,
        grid=(GRID,),
        in_specs=[
            pl.BlockSpec((NC, RB, 8), lambda i: (0, i, 0)),
            pl.BlockSpec((1, 2), lambda i: (0, 0)),
            pl.BlockSpec((2, 4), lambda i: (0, 0)),
            pl.BlockSpec((1, 4), lambda i: (0, 0)),
        ],
        out_specs=[
            pl.BlockSpec((RB, 2), lambda i: (i, 0)),
            pl.BlockSpec((RB, 4), lambda i: (i, 0)),
        ],
        out_shape=[
            jax.ShapeDtypeStruct((NP, 2), jnp.float32),
            jax.ShapeDtypeStruct((NP, 4), jnp.float32),
        ],
    )(acc, b3, wc, bc)


def _make_edge_kernel(p, f, ep):
    chunks = ep // (NW * EB)
    ew = chunks * EB
    mesh = plsc.VectorSubcoreMesh(core_axis_name="c", subcore_axis_name="s")

    @functools.partial(
        pl.kernel,
        mesh=mesh,
        out_type=jax.ShapeDtypeStruct((NC, NP, p), jnp.float32),
        compiler_params=pltpu.CompilerParams(
            needs_layout_passes=False, use_tc_tiling_on_sc=False),
        scratch_types=[
            pltpu.VMEM((EB,), jnp.int32),
            pltpu.VMEM((EB,), jnp.int32),
            pltpu.VMEM((EB, p), jnp.float32),
            pltpu.VMEM((EB, p), jnp.float32),
            pltpu.VMEM((EB, p), jnp.float32),
            pltpu.VMEM((128,), jnp.float32),
            pltpu.VMEM((128, p), jnp.float32),
            pltpu.VMEM_SHARED((NP, p), jnp.float32),
            pltpu.SemaphoreType.DMA,
            pltpu.SemaphoreType.DMA,
        ],
    )
    def k(src_hbm, dst_hbm, table_hbm, cvec_hbm, zs_hbm, out_hbm,
          src_v, dst_v, rows_s, rows_d, out_v, cvec_v, zbuf, acc,
          sem1, sem2):
        ci = lax.axis_index("c")
        si = lax.axis_index("s")
        wid = ci * NS + si
        iota = lax.iota(jnp.int32, 16)
        zf = jnp.zeros((16,), jnp.float32)

        pltpu.sync_copy(zs_hbm, zbuf)
        r0 = si * SUBROWS

        def zacc(j, carry):
            pltpu.sync_copy(zbuf, acc.at[pl.ds(r0 + j * 128, 128)])
            return carry
        lax.fori_loop(0, SUBROWS // 128, zacc, 0)
        plsc.subcore_barrier()

        shift = {8: 3, 4: 2}[p]

        def zout(j, carry):
            flat = j * 16 + iota
            plsc.store_scatter(
                out_v,
                [lax.shift_right_logical(flat, shift),
                 lax.bitwise_and(flat, p - 1)],
                zf)
            return carry
        lax.fori_loop(0, EB * p // 16, zout, 0)

        pltpu.sync_copy(cvec_hbm.at[0], cvec_v)
        cv = cvec_v[pl.ds(0, 16)]
        cols = [jnp.full((16,), j, jnp.int32) for j in range(f + 2)]
        wbase = wid * ew

        def chunk(kk, carry):
            off = wbase + kk * EB
            pltpu.sync_copy(src_hbm.at[pl.ds(off, EB)], src_v)
            pltpu.sync_copy(dst_hbm.at[pl.ds(off, EB)], dst_v)
            g1 = pltpu.async_copy(table_hbm.at[src_v], rows_s, sem1)
            g2 = pltpu.async_copy(table_hbm.at[dst_v], rows_d, sem2)
            g1.wait()
            g2.wait()

            def group(g, c2):
                ridx = g * 16 + iota
                s = plsc.load_gather(rows_s, [ridx, cols[0]])
                dd = plsc.load_gather(rows_d, [ridx, cols[f + 1]])
                t = s + dd
                e = jnp.maximum(t, NEG * t)
                w = jnp.exp(e - cv)
                plsc.store_scatter(out_v, [ridx, cols[0]], w)
                for j in range(f):
                    hj = plsc.load_gather(rows_s, [ridx, cols[1 + j]])
                    plsc.store_scatter(out_v, [ridx, cols[1 + j]], w * hj)
                return c2
            lax.fori_loop(0, EB // 16, group, 0)
            pltpu.sync_copy(out_v, acc.at[dst_v], add=True)
            return carry
        lax.fori_loop(0, chunks, chunk, 0)
        plsc.subcore_barrier()

        pltpu.sync_copy(acc.at[pl.ds(r0, SUBROWS)],
                        out_hbm.at[ci, pl.ds(r0, SUBROWS)])

    return k


def kernel(x, edge_index, W1, a1s, a1d, b1, W2, a2s, a2d, b2,
           W3, a3s, a3d, b3, Wc, bc):
    e = edge_index.shape[1]
    ne = e + N
    ep = ((ne + NW * EB - 1) // (NW * EB)) * (NW * EB)

    loops = jnp.arange(N, dtype=edge_index.dtype)
    padi = jnp.full((ep - ne,), NP - 1, edge_index.dtype)
    src = jnp.concatenate([edge_index[0], loops, padi])
    dst = jnp.concatenate([edge_index[1], loops, padi])
    xp = jnp.pad(x, ((0, NP - N), (0, 0)))

    a1 = jnp.stack([a1s, a1d], axis=1)
    a2 = jnp.stack([a2s, a2d], axis=1)
    a3 = jnp.stack([a3s, a3d], axis=1)

    edge8 = _make_edge_kernel(8, 4, ep)
    edge8b = _make_edge_kernel(8, 2, ep)
    zs8 = jnp.zeros((128, 8), jnp.float32)

    table1, cvec1 = _prep1_call(xp, W1, a1)
    acc1 = edge8(src, dst, table1, cvec1, zs8)
    h1, table2, cvec2 = _mid_call(acc1, b1.reshape(1, 4), W2, a2, 4, 8)
    acc2 = edge8(src, dst, table2, cvec2, zs8)
    h2, table3, cvec3 = _mid_call(acc2, b2.reshape(1, 4), W3, a3, 2, 8)
    acc3 = edge8b(src, dst, table3, cvec3, zs8)
    h3, out = _final_call(acc3, b3.reshape(1, 2), Wc, bc.reshape(1, 4))

    return (h1[:N], h2[:N], h3[:N], out[:N])


# speedup vs baseline: 90.4740x; 1.9690x over previous
import functools

import jax
import jax.numpy as jnp
from jax import lax
from jax.experimental import pallas as pl
from jax.experimental.pallas import tpu as pltpu
from jax.experimental.pallas import tpu_sc as plsc

N = 100000
NP = 102400
RB = 2048
GRID = NP // RB
NEG = 0.2
EPS = 1e-16

NC, NS = 2, 16
NW = NC * NS
EB = 512
SUPER = 8
SUBROWS = NP // NS


def _prep1_body(x_ref, w_ref, a_ref, table_ref, cvec_ref, mx_ref):
    i = pl.program_id(0)
    h = jnp.dot(x_ref[...], w_ref[...], preferred_element_type=jnp.float32)
    sd = jnp.dot(h, a_ref[...], preferred_element_type=jnp.float32)
    table_ref[...] = jnp.concatenate(
        [sd[:, 0:1], h, sd[:, 1:2], jnp.zeros((RB, 2), jnp.float32)], axis=1)
    bs = jnp.max(sd[:, 0])
    bd = jnp.max(sd[:, 1])
    ps = jnp.where(i == 0, -jnp.inf, mx_ref[0])
    pd = jnp.where(i == 0, -jnp.inf, mx_ref[1])
    mx_ref[0] = jnp.maximum(ps, bs)
    mx_ref[1] = jnp.maximum(pd, bd)

    @pl.when(i == GRID - 1)
    def _():
        t = mx_ref[0] + mx_ref[1]
        c = jnp.maximum(t, NEG * t)
        cvec_ref[...] = jnp.full((8, 128), c, jnp.float32)


def _prep1_call(xp, w1, a1):
    return pl.pallas_call(
        _prep1_body,
        grid=(GRID,),
        in_specs=[
            pl.BlockSpec((RB, 34), lambda i: (i, 0)),
            pl.BlockSpec((34, 4), lambda i: (0, 0)),
            pl.BlockSpec((4, 2), lambda i: (0, 0)),
        ],
        out_specs=[
            pl.BlockSpec((RB, 8), lambda i: (i, 0)),
            pl.BlockSpec((8, 128), lambda i: (0, 0)),
        ],
        out_shape=[
            jax.ShapeDtypeStruct((NP, 8), jnp.float32),
            jax.ShapeDtypeStruct((8, 128), jnp.float32),
        ],
        scratch_shapes=[pltpu.SMEM((2,), jnp.float32)],
    )(xp, w1, a1)


def _mid_body(acc_ref, b_ref, w_ref, a_ref, h_ref, table_ref, cvec_ref,
              mx_ref, *, fn, pout):
    i = pl.program_id(0)
    a = acc_ref[0] + acc_ref[1]
    o = a[:, 1:5] / (a[:, 0:1] + EPS) + b_ref[...]
    hl = jnp.tanh(o)
    h_ref[...] = hl
    hn = jnp.dot(hl, w_ref[...], preferred_element_type=jnp.float32)
    sd = jnp.dot(hn, a_ref[...], preferred_element_type=jnp.float32)
    parts = [sd[:, 0:1], hn, sd[:, 1:2]]
    if pout > fn + 2:
        parts.append(jnp.zeros((RB, pout - fn - 2), jnp.float32))
    table_ref[...] = jnp.concatenate(parts, axis=1)
    bs = jnp.max(sd[:, 0])
    bd = jnp.max(sd[:, 1])
    ps = jnp.where(i == 0, -jnp.inf, mx_ref[0])
    pd = jnp.where(i == 0, -jnp.inf, mx_ref[1])
    mx_ref[0] = jnp.maximum(ps, bs)
    mx_ref[1] = jnp.maximum(pd, bd)

    @pl.when(i == GRID - 1)
    def _():
        t = mx_ref[0] + mx_ref[1]
        c = jnp.maximum(t, NEG * t)
        cvec_ref[...] = jnp.full((8, 128), c, jnp.float32)


def _mid_call(acc, b, w, a, fn, pout):
    return pl.pallas_call(
        functools.partial(_mid_body, fn=fn, pout=pout),
        grid=(GRID,),
        in_specs=[
            pl.BlockSpec((NC, RB, 8), lambda i: (0, i, 0)),
            pl.BlockSpec((1, 4), lambda i: (0, 0)),
            pl.BlockSpec((4, fn), lambda i: (0, 0)),
            pl.BlockSpec((fn, 2), lambda i: (0, 0)),
        ],
        out_specs=[
            pl.BlockSpec((RB, 4), lambda i: (i, 0)),
            pl.BlockSpec((RB, pout), lambda i: (i, 0)),
            pl.BlockSpec((8, 128), lambda i: (0, 0)),
        ],
        out_shape=[
            jax.ShapeDtypeStruct((NP, 4), jnp.float32),
            jax.ShapeDtypeStruct((NP, pout), jnp.float32),
            jax.ShapeDtypeStruct((8, 128), jnp.float32),
        ],
        scratch_shapes=[pltpu.SMEM((2,), jnp.float32)],
    )(acc, b, w, a)


def _final_body(acc_ref, b_ref, wc_ref, bc_ref, h_ref, out_ref):
    a = acc_ref[0] + acc_ref[1]
    o = a[:, 1:3] / (a[:, 0:1] + EPS) + b_ref[...]
    h3 = jnp.tanh(o)
    h_ref[...] = h3
    out_ref[...] = (
        jnp.dot(h3, wc_ref[...], preferred_element_type=jnp.float32)
        + bc_ref[...])


def _final_call(acc, b3, wc, bc):
    return pl.pallas_call(
        _final_body,
        grid=(GRID,),
        in_specs=[
            pl.BlockSpec((NC, RB, 8), lambda i: (0, i, 0)),
            pl.BlockSpec((1, 2), lambda i: (0, 0)),
            pl.BlockSpec((2, 4), lambda i: (0, 0)),
            pl.BlockSpec((1, 4), lambda i: (0, 0)),
        ],
        out_specs=[
            pl.BlockSpec((RB, 2), lambda i: (i, 0)),
            pl.BlockSpec((RB, 4), lambda i: (i, 0)),
        ],
        out_shape=[
            jax.ShapeDtypeStruct((NP, 2), jnp.float32),
            jax.ShapeDtypeStruct((NP, 4), jnp.float32),
        ],
    )(acc, b3, wc, bc)


def _make_edge_kernel(p, f, ep):
    chunks = ep // (NW * EB)
    subc = EB // 128
    super_rows = SUPER * subc
    mesh = plsc.VectorSubcoreMesh(core_axis_name="c", subcore_axis_name="s")

    @functools.partial(
        pl.kernel,
        mesh=mesh,
        out_type=jax.ShapeDtypeStruct((NC, NP, p), jnp.float32),
        compiler_params=pltpu.CompilerParams(
            needs_layout_passes=False, use_tc_tiling_on_sc=False),
        scratch_types=[
            pltpu.VMEM((super_rows, 128), jnp.int32),
            pltpu.VMEM((super_rows, 128), jnp.int32),
            pltpu.VMEM((EB, p), jnp.float32),
            pltpu.VMEM((EB, p), jnp.float32),
            pltpu.VMEM((EB, p), jnp.float32),
            pltpu.VMEM((EB, p), jnp.float32),
            pltpu.VMEM((EB, p), jnp.float32),
            pltpu.VMEM((EB, p), jnp.float32),
            pltpu.VMEM((128,), jnp.float32),
            pltpu.VMEM((128, p), jnp.float32),
            pltpu.VMEM_SHARED((NP, p), jnp.float32),
            pltpu.SemaphoreType.DMA,
            pltpu.SemaphoreType.DMA,
            pltpu.SemaphoreType.DMA,
            pltpu.SemaphoreType.DMA,
        ],
    )
    def k(src_hbm, dst_hbm, table_hbm, cvec_hbm, zs_hbm, out_hbm,
          src_sv, dst_sv, rs0, rd0, rs1, rd1, o0, o1, cvec_v, zbuf, acc,
          gs0, gs1, ss0, ss1):
        ci = lax.axis_index("c")
        si = lax.axis_index("s")
        wid = ci * NS + si
        iota = lax.iota(jnp.int32, 16)
        rs = (rs0, rs1)
        rd = (rd0, rd1)
        out = (o0, o1)
        gs = (gs0, gs1)
        ss = (ss0, ss1)

        pltpu.sync_copy(zs_hbm, zbuf)
        r0 = si * SUBROWS

        def zacc(j, carry):
            pltpu.sync_copy(zbuf, acc.at[pl.ds(r0 + j * 128, 128)])
            return carry
        lax.fori_loop(0, SUBROWS // 128, zacc, 0)
        for b in (0, 1):
            for q in range(subc):
                pltpu.sync_copy(zs_hbm, out[b].at[pl.ds(q * 128, 128)])
        plsc.subcore_barrier()

        pltpu.sync_copy(cvec_hbm.at[0], cvec_v)
        cv = cvec_v[pl.ds(0, 16)]
        cols = [jnp.full((16,), j, jnp.int32) for j in range(f + 2)]
        rbase = wid * chunks * subc

        def issue_gathers(kk, b):
            for j in range(subc):
                row = lax.rem(kk, SUPER) * subc + j
                pltpu.async_copy(table_hbm.at[src_sv.at[row]],
                                 rs[b].at[pl.ds(j * 128, 128)], gs[b])
                pltpu.async_copy(table_hbm.at[dst_sv.at[row]],
                                 rd[b].at[pl.ds(j * 128, 128)], gs[b])

        def drain_gathers(b):
            pltpu.make_async_copy(
                table_hbm.at[pl.ds(0, EB)], rs[b], gs[b]).wait()
            pltpu.make_async_copy(
                table_hbm.at[pl.ds(0, EB)], rd[b], gs[b]).wait()

        def issue_scatters(kk, b):
            for j in range(subc):
                row = lax.rem(kk, SUPER) * subc + j
                pltpu.async_copy(out[b].at[pl.ds(j * 128, 128)],
                                 acc.at[dst_sv.at[row]], ss[b], add=True)

        def drain_scatters(b):
            pltpu.make_async_copy(out[b], acc.at[pl.ds(0, EB)], ss[b]).wait()

        def load_super(kk):
            base = rbase + lax.shift_right_logical(kk, 3) * super_rows
            pltpu.sync_copy(src_hbm.at[pl.ds(base, super_rows)], src_sv)
            pltpu.sync_copy(dst_hbm.at[pl.ds(base, super_rows)], dst_sv)

        def compute(b):
            def group(g, c2):
                ridx = g * 16 + iota
                s = plsc.load_gather(rs[b], [ridx, cols[0]])
                dd = plsc.load_gather(rd[b], [ridx, cols[f + 1]])
                t = s + dd
                e = jnp.maximum(t, NEG * t)
                w = jnp.exp(e - cv)
                plsc.store_scatter(out[b], [ridx, cols[0]], w)
                for j in range(f):
                    hj = plsc.load_gather(rs[b], [ridx, cols[1 + j]])
                    plsc.store_scatter(out[b], [ridx, cols[1 + j]], w * hj)
                return c2
            lax.fori_loop(0, EB // 16, group, 0)

        def pair(i2, carry):
            for b in (0, 1):
                kk = i2 * 2 + b
                if b == 0:
                    @pl.when(lax.rem(kk, SUPER) == 0)
                    def _():
                        load_super(kk)
                        issue_gathers(kk, 0)
                    drain_gathers(0)
                    issue_gathers(kk + 1, 1)
                else:
                    drain_gathers(1)

                    @pl.when(lax.rem(kk + 1, SUPER) != 0)
                    def _():
                        issue_gathers(kk + 1, 0)

                @pl.when(kk >= 2)
                def _():
                    drain_scatters(b)
                compute(b)
                issue_scatters(kk, b)
            return carry
        lax.fori_loop(0, chunks // 2, pair, 0)
        drain_scatters(0)
        drain_scatters(1)
        plsc.subcore_barrier()

        pltpu.sync_copy(acc.at[pl.ds(r0, SUBROWS)],
                        out_hbm.at[ci, pl.ds(r0, SUBROWS)])

    return k


def kernel(x, edge_index, W1, a1s, a1d, b1, W2, a2s, a2d, b2,
           W3, a3s, a3d, b3, Wc, bc):
    e = edge_index.shape[1]
    ne = e + N
    blk = NW * EB * SUPER
    ep = ((ne + blk - 1) // blk) * blk

    loops = jnp.arange(N, dtype=edge_index.dtype)
    padi = jnp.full((ep - ne,), NP - 1, edge_index.dtype)
    src = jnp.concatenate([edge_index[0], loops, padi]).reshape(-1, 128)
    dst = jnp.concatenate([edge_index[1], loops, padi]).reshape(-1, 128)
    xp = jnp.pad(x, ((0, NP - N), (0, 0)))

    a1 = jnp.stack([a1s, a1d], axis=1)
    a2 = jnp.stack([a2s, a2d], axis=1)
    a3 = jnp.stack([a3s, a3d], axis=1)

    edge8 = _make_edge_kernel(8, 4, ep)
    edge8b = _make_edge_kernel(8, 2, ep)
    zs8 = jnp.zeros((128, 8), jnp.float32)

    table1, cvec1 = _prep1_call(xp, W1, a1)
    acc1 = edge8(src, dst, table1, cvec1, zs8)
    h1, table2, cvec2 = _mid_call(acc1, b1.reshape(1, 4), W2, a2, 4, 8)
    acc2 = edge8(src, dst, table2, cvec2, zs8)
    h2, table3, cvec3 = _mid_call(acc2, b2.reshape(1, 4), W3, a3, 2, 8)
    acc3 = edge8b(src, dst, table3, cvec3, zs8)
    h3, out = _final_call(acc3, b3.reshape(1, 2), Wc, bc.reshape(1, 4))

    return (h1[:N], h2[:N], h3[:N], out[:N])


# speedup vs baseline: 93.8545x; 1.0374x over previous
import functools

import jax
import jax.numpy as jnp
from jax import lax
from jax.experimental import pallas as pl
from jax.experimental.pallas import tpu as pltpu
from jax.experimental.pallas import tpu_sc as plsc

N = 100000
NP = 102400
RB = 10240
GRID = NP // RB
NEG = 0.2
EPS = 1e-16

NC, NS = 2, 16
NW = NC * NS
EB = 512
SUPER = 8
SUBROWS = NP // NS


def _prep1_body(x_ref, w_ref, a_ref, table_ref, cvec_ref, mx_ref):
    i = pl.program_id(0)
    h = jnp.dot(x_ref[...], w_ref[...], preferred_element_type=jnp.float32)
    sd = jnp.dot(h, a_ref[...], preferred_element_type=jnp.float32)
    table_ref[...] = jnp.concatenate(
        [sd[:, 0:1], h, sd[:, 1:2], jnp.zeros((RB, 2), jnp.float32)], axis=1)
    bs = jnp.max(sd[:, 0])
    bd = jnp.max(sd[:, 1])
    ps = jnp.where(i == 0, -jnp.inf, mx_ref[0])
    pd = jnp.where(i == 0, -jnp.inf, mx_ref[1])
    mx_ref[0] = jnp.maximum(ps, bs)
    mx_ref[1] = jnp.maximum(pd, bd)

    @pl.when(i == GRID - 1)
    def _():
        t = mx_ref[0] + mx_ref[1]
        c = jnp.maximum(t, NEG * t)
        cvec_ref[...] = jnp.full((8, 128), c, jnp.float32)


def _prep1_call(xp, w1, a1):
    return pl.pallas_call(
        _prep1_body,
        grid=(GRID,),
        in_specs=[
            pl.BlockSpec((RB, 34), lambda i: (i, 0)),
            pl.BlockSpec((34, 4), lambda i: (0, 0)),
            pl.BlockSpec((4, 2), lambda i: (0, 0)),
        ],
        out_specs=[
            pl.BlockSpec((RB, 8), lambda i: (i, 0)),
            pl.BlockSpec((8, 128), lambda i: (0, 0)),
        ],
        out_shape=[
            jax.ShapeDtypeStruct((NP, 8), jnp.float32),
            jax.ShapeDtypeStruct((8, 128), jnp.float32),
        ],
        scratch_shapes=[pltpu.SMEM((2,), jnp.float32)],
    )(xp, w1, a1)


def _mid_body(acc_ref, b_ref, w_ref, a_ref, h_ref, table_ref, cvec_ref,
              mx_ref, *, fn, pout):
    i = pl.program_id(0)
    a = acc_ref[0] + acc_ref[1]
    o = a[:, 1:5] / (a[:, 0:1] + EPS) + b_ref[...]
    hl = jnp.tanh(o)
    h_ref[...] = hl
    hn = jnp.dot(hl, w_ref[...], preferred_element_type=jnp.float32)
    sd = jnp.dot(hn, a_ref[...], preferred_element_type=jnp.float32)
    parts = [sd[:, 0:1], hn, sd[:, 1:2]]
    if pout > fn + 2:
        parts.append(jnp.zeros((RB, pout - fn - 2), jnp.float32))
    table_ref[...] = jnp.concatenate(parts, axis=1)
    bs = jnp.max(sd[:, 0])
    bd = jnp.max(sd[:, 1])
    ps = jnp.where(i == 0, -jnp.inf, mx_ref[0])
    pd = jnp.where(i == 0, -jnp.inf, mx_ref[1])
    mx_ref[0] = jnp.maximum(ps, bs)
    mx_ref[1] = jnp.maximum(pd, bd)

    @pl.when(i == GRID - 1)
    def _():
        t = mx_ref[0] + mx_ref[1]
        c = jnp.maximum(t, NEG * t)
        cvec_ref[...] = jnp.full((8, 128), c, jnp.float32)


def _mid_call(acc, b, w, a, fn, pout):
    return pl.pallas_call(
        functools.partial(_mid_body, fn=fn, pout=pout),
        grid=(GRID,),
        in_specs=[
            pl.BlockSpec((NC, RB, 8), lambda i: (0, i, 0)),
            pl.BlockSpec((1, 4), lambda i: (0, 0)),
            pl.BlockSpec((4, fn), lambda i: (0, 0)),
            pl.BlockSpec((fn, 2), lambda i: (0, 0)),
        ],
        out_specs=[
            pl.BlockSpec((RB, 4), lambda i: (i, 0)),
            pl.BlockSpec((RB, pout), lambda i: (i, 0)),
            pl.BlockSpec((8, 128), lambda i: (0, 0)),
        ],
        out_shape=[
            jax.ShapeDtypeStruct((NP, 4), jnp.float32),
            jax.ShapeDtypeStruct((NP, pout), jnp.float32),
            jax.ShapeDtypeStruct((8, 128), jnp.float32),
        ],
        scratch_shapes=[pltpu.SMEM((2,), jnp.float32)],
    )(acc, b, w, a)


def _final_body(acc_ref, b_ref, wc_ref, bc_ref, h_ref, out_ref):
    a = acc_ref[0] + acc_ref[1]
    o = a[:, 1:3] / (a[:, 0:1] + EPS) + b_ref[...]
    h3 = jnp.tanh(o)
    h_ref[...] = h3
    out_ref[...] = (
        jnp.dot(h3, wc_ref[...], preferred_element_type=jnp.float32)
        + bc_ref[...])


def _final_call(acc, b3, wc, bc):
    return pl.pallas_call(
        _final_body,
        grid=(GRID,),
        in_specs=[
            pl.BlockSpec((NC, RB, 8), lambda i: (0, i, 0)),
            pl.BlockSpec((1, 2), lambda i: (0, 0)),
            pl.BlockSpec((2, 4), lambda i: (0, 0)),
            pl.BlockSpec((1, 4), lambda i: (0, 0)),
        ],
        out_specs=[
            pl.BlockSpec((RB, 2), lambda i: (i, 0)),
            pl.BlockSpec((RB, 4), lambda i: (i, 0)),
        ],
        out_shape=[
            jax.ShapeDtypeStruct((NP, 2), jnp.float32),
            jax.ShapeDtypeStruct((NP, 4), jnp.float32),
        ],
    )(acc, b3, wc, bc)


def _make_edge_kernel(p, f, ep):
    chunks = ep // (NW * EB)
    subc = EB // 128
    super_rows = SUPER * subc
    mesh = plsc.VectorSubcoreMesh(core_axis_name="c", subcore_axis_name="s")

    @functools.partial(
        pl.kernel,
        mesh=mesh,
        out_type=jax.ShapeDtypeStruct((NC, NP, p), jnp.float32),
        compiler_params=pltpu.CompilerParams(
            needs_layout_passes=False, use_tc_tiling_on_sc=False),
        scratch_types=[
            pltpu.VMEM((super_rows, 128), jnp.int32),
            pltpu.VMEM((super_rows, 128), jnp.int32),
            pltpu.VMEM((EB, p), jnp.float32),
            pltpu.VMEM((EB, p), jnp.float32),
            pltpu.VMEM((EB, p), jnp.float32),
            pltpu.VMEM((EB, p), jnp.float32),
            pltpu.VMEM((EB, p), jnp.float32),
            pltpu.VMEM((EB, p), jnp.float32),
            pltpu.VMEM((128,), jnp.float32),
            pltpu.VMEM((128, p), jnp.float32),
            pltpu.VMEM_SHARED((NP, p), jnp.float32),
            pltpu.SemaphoreType.DMA,
            pltpu.SemaphoreType.DMA,
            pltpu.SemaphoreType.DMA,
            pltpu.SemaphoreType.DMA,
        ],
    )
    def k(src_hbm, dst_hbm, table_hbm, cvec_hbm, zs_hbm, out_hbm,
          src_sv, dst_sv, rs0, rd0, rs1, rd1, o0, o1, cvec_v, zbuf, acc,
          gs0, gs1, ss0, ss1):
        ci = lax.axis_index("c")
        si = lax.axis_index("s")
        wid = ci * NS + si
        iota = lax.iota(jnp.int32, 16)
        rs = (rs0, rs1)
        rd = (rd0, rd1)
        out = (o0, o1)
        gs = (gs0, gs1)
        ss = (ss0, ss1)

        pltpu.sync_copy(zs_hbm, zbuf)
        r0 = si * SUBROWS

        def zacc(j, carry):
            pltpu.sync_copy(zbuf, acc.at[pl.ds(r0 + j * 128, 128)])
            return carry
        lax.fori_loop(0, SUBROWS // 128, zacc, 0)
        for b in (0, 1):
            for q in range(subc):
                pltpu.sync_copy(zs_hbm, out[b].at[pl.ds(q * 128, 128)])
        plsc.subcore_barrier()

        pltpu.sync_copy(cvec_hbm.at[0], cvec_v)
        cv = cvec_v[pl.ds(0, 16)]
        cols = [jnp.full((16,), j, jnp.int32) for j in range(f + 2)]
        rbase = wid * chunks * subc

        def issue_gathers(kk, b):
            for j in range(subc):
                row = lax.rem(kk, SUPER) * subc + j
                pltpu.async_copy(table_hbm.at[src_sv.at[row]],
                                 rs[b].at[pl.ds(j * 128, 128)], gs[b])
                pltpu.async_copy(table_hbm.at[dst_sv.at[row]],
                                 rd[b].at[pl.ds(j * 128, 128)], gs[b])

        def drain_gathers(b):
            pltpu.make_async_copy(
                table_hbm.at[pl.ds(0, EB)], rs[b], gs[b]).wait()
            pltpu.make_async_copy(
                table_hbm.at[pl.ds(0, EB)], rd[b], gs[b]).wait()

        def issue_scatters(kk, b):
            for j in range(subc):
                row = lax.rem(kk, SUPER) * subc + j
                pltpu.async_copy(out[b].at[pl.ds(j * 128, 128)],
                                 acc.at[dst_sv.at[row]], ss[b], add=True)

        def drain_scatters(b):
            pltpu.make_async_copy(out[b], acc.at[pl.ds(0, EB)], ss[b]).wait()

        def load_super(kk):
            base = rbase + lax.shift_right_logical(kk, 3) * super_rows
            pltpu.sync_copy(src_hbm.at[pl.ds(base, super_rows)], src_sv)
            pltpu.sync_copy(dst_hbm.at[pl.ds(base, super_rows)], dst_sv)

        def compute(b):
            def group(g, c2):
                ridx = g * 16 + iota
                s = plsc.load_gather(rs[b], [ridx, cols[0]])
                dd = plsc.load_gather(rd[b], [ridx, cols[f + 1]])
                t = s + dd
                e = jnp.maximum(t, NEG * t)
                w = jnp.exp(e - cv)
                plsc.store_scatter(out[b], [ridx, cols[0]], w)
                for j in range(f):
                    hj = plsc.load_gather(rs[b], [ridx, cols[1 + j]])
                    plsc.store_scatter(out[b], [ridx, cols[1 + j]], w * hj)
                return c2
            lax.fori_loop(0, EB // 16, group, 0)

        def pair(i2, carry):
            for b in (0, 1):
                kk = i2 * 2 + b
                if b == 0:
                    @pl.when(lax.rem(kk, SUPER) == 0)
                    def _():
                        load_super(kk)
                        issue_gathers(kk, 0)
                    drain_gathers(0)
                    issue_gathers(kk + 1, 1)
                else:
                    drain_gathers(1)

                    @pl.when(lax.rem(kk + 1, SUPER) != 0)
                    def _():
                        issue_gathers(kk + 1, 0)

                @pl.when(kk >= 2)
                def _():
                    drain_scatters(b)
                compute(b)
                issue_scatters(kk, b)
            return carry
        lax.fori_loop(0, chunks // 2, pair, 0)
        drain_scatters(0)
        drain_scatters(1)
        plsc.subcore_barrier()

        pltpu.sync_copy(acc.at[pl.ds(r0, SUBROWS)],
                        out_hbm.at[ci, pl.ds(r0, SUBROWS)])

    return k


def kernel(x, edge_index, W1, a1s, a1d, b1, W2, a2s, a2d, b2,
           W3, a3s, a3d, b3, Wc, bc):
    e = edge_index.shape[1]
    ne = e + N
    blk = NW * EB * SUPER
    ep = ((ne + blk - 1) // blk) * blk

    loops = jnp.arange(N, dtype=edge_index.dtype)
    padi = jnp.full((ep - ne,), NP - 1, edge_index.dtype)
    src = jnp.concatenate([edge_index[0], loops, padi]).reshape(-1, 128)
    dst = jnp.concatenate([edge_index[1], loops, padi]).reshape(-1, 128)
    xp = jnp.pad(x, ((0, NP - N), (0, 0)))

    a1 = jnp.stack([a1s, a1d], axis=1)
    a2 = jnp.stack([a2s, a2d], axis=1)
    a3 = jnp.stack([a3s, a3d], axis=1)

    edge8 = _make_edge_kernel(8, 4, ep)
    edge8b = _make_edge_kernel(8, 2, ep)
    zs8 = jnp.zeros((128, 8), jnp.float32)

    table1, cvec1 = _prep1_call(xp, W1, a1)
    acc1 = edge8(src, dst, table1, cvec1, zs8)
    h1, table2, cvec2 = _mid_call(acc1, b1.reshape(1, 4), W2, a2, 4, 8)
    acc2 = edge8(src, dst, table2, cvec2, zs8)
    h2, table3, cvec3 = _mid_call(acc2, b2.reshape(1, 4), W3, a3, 2, 8)
    acc3 = edge8b(src, dst, table3, cvec3, zs8)
    h3, out = _final_call(acc3, b3.reshape(1, 2), Wc, bc.reshape(1, 4))

    return (h1[:N], h2[:N], h3[:N], out[:N])


# speedup vs baseline: 100.5628x; 1.0715x over previous
import functools

import jax
import jax.numpy as jnp
from jax import lax
from jax.experimental import pallas as pl
from jax.experimental.pallas import tpu as pltpu
from jax.experimental.pallas import tpu_sc as plsc

N = 100000
NP = 100000
RB = 10000
GRID = NP // RB
NEG = 0.2
EPS = 1e-16

NC, NS = 2, 16
NW = NC * NS
EB = 512
SUPER = 8
SUBROWS = NP // NS


def _prep1_body(x_ref, w_ref, a_ref, table_ref, cvec_ref, mx_ref):
    i = pl.program_id(0)
    h = jnp.dot(x_ref[...], w_ref[...], preferred_element_type=jnp.float32)
    sd = jnp.dot(h, a_ref[...], preferred_element_type=jnp.float32)
    table_ref[...] = jnp.concatenate(
        [sd[:, 0:1], h, sd[:, 1:2], jnp.zeros((RB, 2), jnp.float32)], axis=1)
    bs = jnp.max(sd[:, 0])
    bd = jnp.max(sd[:, 1])
    ps = jnp.where(i == 0, -jnp.inf, mx_ref[0])
    pd = jnp.where(i == 0, -jnp.inf, mx_ref[1])
    mx_ref[0] = jnp.maximum(ps, bs)
    mx_ref[1] = jnp.maximum(pd, bd)

    @pl.when(i == GRID - 1)
    def _():
        t = mx_ref[0] + mx_ref[1]
        c = jnp.maximum(t, NEG * t)
        cvec_ref[...] = jnp.full((8, 128), c, jnp.float32)


def _prep1_call(xp, w1, a1):
    return pl.pallas_call(
        _prep1_body,
        grid=(GRID,),
        in_specs=[
            pl.BlockSpec((RB, 34), lambda i: (i, 0)),
            pl.BlockSpec((34, 4), lambda i: (0, 0)),
            pl.BlockSpec((4, 2), lambda i: (0, 0)),
        ],
        out_specs=[
            pl.BlockSpec((RB, 8), lambda i: (i, 0)),
            pl.BlockSpec((8, 128), lambda i: (0, 0)),
        ],
        out_shape=[
            jax.ShapeDtypeStruct((NP, 8), jnp.float32),
            jax.ShapeDtypeStruct((8, 128), jnp.float32),
        ],
        scratch_shapes=[pltpu.SMEM((2,), jnp.float32)],
    )(xp, w1, a1)


def _mid_body(acc_ref, b_ref, w_ref, a_ref, h_ref, table_ref, cvec_ref,
              mx_ref, *, fn, pout):
    i = pl.program_id(0)
    a = acc_ref[0] + acc_ref[1]
    o = a[:, 1:5] / (a[:, 0:1] + EPS) + b_ref[...]
    hl = jnp.tanh(o)
    h_ref[...] = hl
    hn = jnp.dot(hl, w_ref[...], preferred_element_type=jnp.float32)
    sd = jnp.dot(hn, a_ref[...], preferred_element_type=jnp.float32)
    parts = [sd[:, 0:1], hn, sd[:, 1:2]]
    if pout > fn + 2:
        parts.append(jnp.zeros((RB, pout - fn - 2), jnp.float32))
    table_ref[...] = jnp.concatenate(parts, axis=1)
    bs = jnp.max(sd[:, 0])
    bd = jnp.max(sd[:, 1])
    ps = jnp.where(i == 0, -jnp.inf, mx_ref[0])
    pd = jnp.where(i == 0, -jnp.inf, mx_ref[1])
    mx_ref[0] = jnp.maximum(ps, bs)
    mx_ref[1] = jnp.maximum(pd, bd)

    @pl.when(i == GRID - 1)
    def _():
        t = mx_ref[0] + mx_ref[1]
        c = jnp.maximum(t, NEG * t)
        cvec_ref[...] = jnp.full((8, 128), c, jnp.float32)


def _mid_call(acc, b, w, a, fn, pout):
    return pl.pallas_call(
        functools.partial(_mid_body, fn=fn, pout=pout),
        grid=(GRID,),
        in_specs=[
            pl.BlockSpec((NC, RB, 8), lambda i: (0, i, 0)),
            pl.BlockSpec((1, 4), lambda i: (0, 0)),
            pl.BlockSpec((4, fn), lambda i: (0, 0)),
            pl.BlockSpec((fn, 2), lambda i: (0, 0)),
        ],
        out_specs=[
            pl.BlockSpec((RB, 4), lambda i: (i, 0)),
            pl.BlockSpec((RB, pout), lambda i: (i, 0)),
            pl.BlockSpec((8, 128), lambda i: (0, 0)),
        ],
        out_shape=[
            jax.ShapeDtypeStruct((NP, 4), jnp.float32),
            jax.ShapeDtypeStruct((NP, pout), jnp.float32),
            jax.ShapeDtypeStruct((8, 128), jnp.float32),
        ],
        scratch_shapes=[pltpu.SMEM((2,), jnp.float32)],
    )(acc, b, w, a)


def _final_body(acc_ref, b_ref, wc_ref, bc_ref, h_ref, out_ref):
    a = acc_ref[0] + acc_ref[1]
    o = a[:, 1:3] / (a[:, 0:1] + EPS) + b_ref[...]
    h3 = jnp.tanh(o)
    h_ref[...] = h3
    out_ref[...] = (
        jnp.dot(h3, wc_ref[...], preferred_element_type=jnp.float32)
        + bc_ref[...])


def _final_call(acc, b3, wc, bc):
    return pl.pallas_call(
        _final_body,
        grid=(GRID,),
        in_specs=[
            pl.BlockSpec((NC, RB, 8), lambda i: (0, i, 0)),
            pl.BlockSpec((1, 2), lambda i: (0, 0)),
            pl.BlockSpec((2, 4), lambda i: (0, 0)),
            pl.BlockSpec((1, 4), lambda i: (0, 0)),
        ],
        out_specs=[
            pl.BlockSpec((RB, 2), lambda i: (i, 0)),
            pl.BlockSpec((RB, 4), lambda i: (i, 0)),
        ],
        out_shape=[
            jax.ShapeDtypeStruct((NP, 2), jnp.float32),
            jax.ShapeDtypeStruct((NP, 4), jnp.float32),
        ],
    )(acc, b3, wc, bc)


def _make_edge_kernel(p, f, ep, ne):
    chunks = ep // (NW * EB)
    subc = EB // 128
    super_rows = SUPER * subc
    mesh = plsc.VectorSubcoreMesh(core_axis_name="c", subcore_axis_name="s")

    @functools.partial(
        pl.kernel,
        mesh=mesh,
        out_type=jax.ShapeDtypeStruct((NC, NP, p), jnp.float32),
        compiler_params=pltpu.CompilerParams(
            needs_layout_passes=False, use_tc_tiling_on_sc=False),
        scratch_types=[
            pltpu.VMEM((super_rows, 128), jnp.int32),
            pltpu.VMEM((super_rows, 128), jnp.int32),
            pltpu.VMEM((EB, p), jnp.float32),
            pltpu.VMEM((EB, p), jnp.float32),
            pltpu.VMEM((EB, p), jnp.float32),
            pltpu.VMEM((EB, p), jnp.float32),
            pltpu.VMEM((EB, p), jnp.float32),
            pltpu.VMEM((EB, p), jnp.float32),
            pltpu.VMEM((128,), jnp.float32),
            pltpu.VMEM((625, p), jnp.float32),
            pltpu.VMEM_SHARED((NP, p), jnp.float32),
            pltpu.SemaphoreType.DMA,
            pltpu.SemaphoreType.DMA,
            pltpu.SemaphoreType.DMA,
            pltpu.SemaphoreType.DMA,
        ],
    )
    def k(src_hbm, dst_hbm, table_hbm, cvec_hbm, zs_hbm, out_hbm,
          src_sv, dst_sv, rs0, rd0, rs1, rd1, o0, o1, cvec_v, zbuf, acc,
          gs0, gs1, ss0, ss1):
        ci = lax.axis_index("c")
        si = lax.axis_index("s")
        wid = ci * NS + si
        iota = lax.iota(jnp.int32, 16)
        rs = (rs0, rs1)
        rd = (rd0, rd1)
        out = (o0, o1)
        gs = (gs0, gs1)
        ss = (ss0, ss1)

        pltpu.sync_copy(zs_hbm, zbuf)
        r0 = si * SUBROWS

        def zacc(j, carry):
            pltpu.sync_copy(zbuf, acc.at[pl.ds(r0 + j * 625, 625)])
            return carry
        lax.fori_loop(0, SUBROWS // 625, zacc, 0)
        for b in (0, 1):
            for q in range(subc):
                pltpu.sync_copy(zs_hbm.at[pl.ds(0, 128)],
                                out[b].at[pl.ds(q * 128, 128)])
        plsc.subcore_barrier()

        pltpu.sync_copy(cvec_hbm.at[0], cvec_v)
        cv = cvec_v[pl.ds(0, 16)]
        cols = [jnp.full((16,), j, jnp.int32) for j in range(f + 2)]
        rbase = wid * chunks * subc

        def issue_gathers(kk, b):
            for j in range(subc):
                row = lax.rem(kk, SUPER) * subc + j
                pltpu.async_copy(table_hbm.at[src_sv.at[row]],
                                 rs[b].at[pl.ds(j * 128, 128)], gs[b])
                pltpu.async_copy(table_hbm.at[dst_sv.at[row]],
                                 rd[b].at[pl.ds(j * 128, 128)], gs[b])

        def drain_gathers(b):
            pltpu.make_async_copy(
                table_hbm.at[pl.ds(0, EB)], rs[b], gs[b]).wait()
            pltpu.make_async_copy(
                table_hbm.at[pl.ds(0, EB)], rd[b], gs[b]).wait()

        def issue_scatters(kk, b):
            for j in range(subc):
                row = lax.rem(kk, SUPER) * subc + j
                pltpu.async_copy(out[b].at[pl.ds(j * 128, 128)],
                                 acc.at[dst_sv.at[row]], ss[b], add=True)

        def drain_scatters(b):
            pltpu.make_async_copy(out[b], acc.at[pl.ds(0, EB)], ss[b]).wait()

        def load_super(kk):
            base = rbase + lax.shift_right_logical(kk, 3) * super_rows
            pltpu.sync_copy(src_hbm.at[pl.ds(base, super_rows)], src_sv)
            pltpu.sync_copy(dst_hbm.at[pl.ds(base, super_rows)], dst_sv)

        ebase0 = wid * chunks * EB

        def compute(kk, b):
            ebase = ebase0 + kk * EB

            def group(g, c2):
                ridx = g * 16 + iota
                s = plsc.load_gather(rs[b], [ridx, cols[0]])
                dd = plsc.load_gather(rd[b], [ridx, cols[f + 1]])
                t = s + dd
                e = jnp.maximum(t, NEG * t)
                w = jnp.exp(e - cv)
                w = jnp.where(ebase + ridx < ne, w, 0.0)
                plsc.store_scatter(out[b], [ridx, cols[0]], w)
                for j in range(f):
                    hj = plsc.load_gather(rs[b], [ridx, cols[1 + j]])
                    plsc.store_scatter(out[b], [ridx, cols[1 + j]], w * hj)
                return c2
            lax.fori_loop(0, EB // 16, group, 0)

        def pair(i2, carry):
            for b in (0, 1):
                kk = i2 * 2 + b
                if b == 0:
                    @pl.when(lax.rem(kk, SUPER) == 0)
                    def _():
                        load_super(kk)
                        issue_gathers(kk, 0)
                    drain_gathers(0)
                    issue_gathers(kk + 1, 1)
                else:
                    drain_gathers(1)

                    @pl.when(lax.rem(kk + 1, SUPER) != 0)
                    def _():
                        issue_gathers(kk + 1, 0)

                @pl.when(kk >= 2)
                def _():
                    drain_scatters(b)
                compute(kk, b)
                issue_scatters(kk, b)
            return carry
        lax.fori_loop(0, chunks // 2, pair, 0)
        drain_scatters(0)
        drain_scatters(1)
        plsc.subcore_barrier()

        pltpu.sync_copy(acc.at[pl.ds(r0, SUBROWS)],
                        out_hbm.at[ci, pl.ds(r0, SUBROWS)])

    return k


def kernel(x, edge_index, W1, a1s, a1d, b1, W2, a2s, a2d, b2,
           W3, a3s, a3d, b3, Wc, bc):
    e = edge_index.shape[1]
    ne = e + N
    blk = NW * EB * SUPER
    ep = ((ne + blk - 1) // blk) * blk

    loops = jnp.arange(N, dtype=edge_index.dtype)
    padi = jnp.zeros((ep - ne,), edge_index.dtype)
    src = jnp.concatenate([edge_index[0], loops, padi]).reshape(-1, 128)
    dst = jnp.concatenate([edge_index[1], loops, padi]).reshape(-1, 128)

    a1 = jnp.stack([a1s, a1d], axis=1)
    a2 = jnp.stack([a2s, a2d], axis=1)
    a3 = jnp.stack([a3s, a3d], axis=1)

    edge8 = _make_edge_kernel(8, 4, ep, ne)
    edge8b = _make_edge_kernel(8, 2, ep, ne)
    zs8 = jnp.zeros((625, 8), jnp.float32)

    table1, cvec1 = _prep1_call(x, W1, a1)
    acc1 = edge8(src, dst, table1, cvec1, zs8)
    h1, table2, cvec2 = _mid_call(acc1, b1.reshape(1, 4), W2, a2, 4, 8)
    acc2 = edge8(src, dst, table2, cvec2, zs8)
    h2, table3, cvec3 = _mid_call(acc2, b2.reshape(1, 4), W3, a3, 2, 8)
    acc3 = edge8b(src, dst, table3, cvec3, zs8)
    h3, out = _final_call(acc3, b3.reshape(1, 2), Wc, bc.reshape(1, 4))

    return (h1, h2, h3, out)
